# baseline (speedup 1.0000x reference)
import jax
import jax.numpy as jnp
from jax import lax
from jax.experimental import pallas as pl
from jax.experimental.pallas import tpu as pltpu
from jax.experimental.pallas import tpu_sc as plsc

N = 10000
E = 320000
D = 128
NC = 2
NS = 16
NW = NC * NS
EW = E // NW
C = 80
NCH = EW // C
RPT = N // NS
BR = 400


def _mm_body(h_ref, w_ref, o_ref):
    o_ref[...] = jnp.dot(h_ref[...], w_ref[...], preferred_element_type=jnp.float32)


def _project(h, W):
    return pl.pallas_call(
        _mm_body,
        grid=(N // BR,),
        in_specs=[pl.BlockSpec((BR, D), lambda i: (i, 0)),
                  pl.BlockSpec((D, D), lambda i: (0, 0))],
        out_specs=pl.BlockSpec((BR, D), lambda i: (i, 0)),
        out_shape=jax.ShapeDtypeStruct((N, D), jnp.float32),
    )(h, W)


def _elu_body(p_ref, o_ref):
    s = p_ref[0] + p_ref[1]
    o_ref[...] = jnp.where(s > 0.0, s, jnp.exp(jnp.minimum(s, 0.0)) - 1.0)


def _sum_body(d_ref, o_ref):
    o_ref[...] = jnp.sum(d_ref[...], axis=0, keepdims=True)


def _denom_total(den):
    out = pl.pallas_call(
        _sum_body,
        out_shape=jax.ShapeDtypeStruct((1, N), jnp.float32),
    )(den)
    return out.reshape(N)


def _finish(parts):
    return pl.pallas_call(
        _elu_body,
        grid=(N // BR,),
        in_specs=[pl.BlockSpec((NC, BR, D), lambda i: (0, i, 0))],
        out_specs=pl.BlockSpec((BR, D), lambda i: (i, 0)),
        out_shape=jax.ShapeDtypeStruct((N, D), jnp.float32),
    )(parts)


def _iota16():
    return lax.iota(jnp.int32, 16)


def _take(x, idx):
    return x.at[idx].get(mode="promise_in_bounds")


def _worker_id():
    return lax.axis_index("s") * NC + lax.axis_index("c")


def _global_max_vec(mx_hbm, mx_v):
    pltpu.sync_copy(mx_hbm, mx_v)
    m = mx_v[0, pl.ds(0, 16)]
    for w in range(1, NW):
        m = jnp.maximum(m, mx_v[w, pl.ds(0, 16)])
    return jnp.full((16,), jnp.max(m), jnp.float32)


def _zero_1d(ref, n):
    zeros = jnp.zeros((16,), jnp.float32)

    def body(i, _):
        plsc.store_scatter(ref, [i * 16 + _iota16()], zeros)
        return 0
    lax.fori_loop(0, n // 16, body, 0)


def _sc_edge_logits_body(z_hbm, src_hbm, dst_hbm, e_hbm, mx_hbm,
                         src_v, dst_v, zs_v, zd_v, e_v, mxo_v, sem):
    base = _worker_id() * EW

    def chunk(i, rm):
        off = base + i * C
        pltpu.sync_copy(src_hbm.at[pl.ds(off, C)], src_v)
        pltpu.sync_copy(dst_hbm.at[pl.ds(off, C)], dst_v)
        d1 = pltpu.async_copy(z_hbm.at[src_v], zs_v, sem)
        d2 = pltpu.async_copy(z_hbm.at[dst_v], zd_v, sem)
        d1.wait()
        d2.wait()

        def group(g, rm):
            rows = g * 16 + _iota16()
            acc = jnp.zeros((16,), jnp.float32)
            for k in range(D):
                col = jnp.full((16,), k, jnp.int32)
                acc = acc + (plsc.load_gather(zs_v, [rows, col]) *
                             plsc.load_gather(zd_v, [rows, col]))
            e16 = jnp.where(acc >= 0.0, acc, 0.2 * acc)
            plsc.store_scatter(e_v, [rows], e16)
            return jnp.maximum(rm, e16)

        rm = lax.fori_loop(0, C // 16, group, rm)
        pltpu.sync_copy(e_v, e_hbm.at[pl.ds(off, C)])
        return rm

    rm = lax.fori_loop(0, NCH, chunk, jnp.full((16,), -3.4e38, jnp.float32))
    mxo_v[...] = rm
    pltpu.sync_copy(mxo_v, mx_hbm.at[_worker_id()])


def _edge_logits(z, src, dst):
    f = pl.kernel(
        _sc_edge_logits_body,
        out_type=[jax.ShapeDtypeStruct((E,), jnp.float32),
                  jax.ShapeDtypeStruct((NW, 16), jnp.float32)],
        mesh=plsc.VectorSubcoreMesh(core_axis_name="c", subcore_axis_name="s"),
        compiler_params=pltpu.CompilerParams(needs_layout_passes=False),
        scratch_types=[
            pltpu.VMEM((C,), jnp.int32),
            pltpu.VMEM((C,), jnp.int32),
            pltpu.VMEM((C, D), jnp.float32),
            pltpu.VMEM((C, D), jnp.float32),
            pltpu.VMEM((C,), jnp.float32),
            pltpu.VMEM((16,), jnp.float32),
            pltpu.SemaphoreType.DMA,
        ],
    )
    return f(z, src, dst)


def _segsum_scatter_add(den_ref, keys, vals):
    sk, sv = plsc.sort_key_val(keys, vals)
    io = _iota16()
    for d in (1, 2, 4, 8):
        idx = jnp.maximum(io - d, 0)
        same = (io >= d) & (_take(sk, idx) == sk)
        sv = sv + jnp.where(same, _take(sv, idx), 0.0)
    nxt = _take(sk, jnp.minimum(io + 1, 15))
    last = (io == 15) | (nxt != sk)
    plsc.addupdate_scatter(den_ref, [sk], sv, mask=last)


def _sc_denom_body(dst_hbm, e_hbm, mx_hbm, den_hbm,
                   dst_v, e_v, mx_v, den_v):
    base = _worker_id() * EW
    gv = _global_max_vec(mx_hbm, mx_v)
    _zero_1d(den_v, N)

    def chunk(i, _):
        off = base + i * C
        pltpu.sync_copy(dst_hbm.at[pl.ds(off, C)], dst_v)
        pltpu.sync_copy(e_hbm.at[pl.ds(off, C)], e_v)

        def group(g, _):
            rows = g * 16 + _iota16()
            d16 = plsc.load_gather(dst_v, [rows])
            e16 = plsc.load_gather(e_v, [rows])
            p16 = jnp.exp(e16 - gv)
            _segsum_scatter_add(den_v, d16, p16)
            return 0
        lax.fori_loop(0, C // 16, group, 0)
        return 0
    lax.fori_loop(0, NCH, chunk, 0)

    pltpu.sync_copy(den_v, den_hbm.at[_worker_id()])


def _denom(dst, e, mx):
    f = pl.kernel(
        _sc_denom_body,
        out_type=[jax.ShapeDtypeStruct((NW, N), jnp.float32)],
        mesh=plsc.VectorSubcoreMesh(core_axis_name="c", subcore_axis_name="s"),
        compiler_params=pltpu.CompilerParams(needs_layout_passes=False),
        scratch_types=[
            pltpu.VMEM((C,), jnp.int32),
            pltpu.VMEM((C,), jnp.float32),
            pltpu.VMEM((NW, 16), jnp.float32),
            pltpu.VMEM((N,), jnp.float32),
        ],
    )
    return f(dst, e, mx)[0]


def _sc_aggregate_body(z_hbm, src_hbm, dst_hbm, e_hbm, mx_hbm, den_hbm, zero_hbm,
                       out_hbm,
                       src_v, dst_v, e_v, a_v, mx_v, den_v, zs_v,
                       hout_sh, sem):
    cid = lax.axis_index("c")
    sid = lax.axis_index("s")
    base = _worker_id() * EW
    gv = _global_max_vec(mx_hbm, mx_v)

    pltpu.sync_copy(den_hbm, den_v)

    s0 = pl.multiple_of(sid * 624, 8)

    @pl.when(sid < NS - 1)
    def _():
        pltpu.sync_copy(zero_hbm.at[pl.ds(s0, 624)],
                        hout_sh.at[pl.ds(s0, 624)])

    @pl.when(sid == NS - 1)
    def _():
        pltpu.sync_copy(zero_hbm.at[pl.ds(624 * (NS - 1), 640)],
                        hout_sh.at[pl.ds(624 * (NS - 1), 640)])
    plsc.subcore_barrier()

    def chunk(i, _):
        off = base + i * C
        pltpu.sync_copy(src_hbm.at[pl.ds(off, C)], src_v)
        pltpu.sync_copy(dst_hbm.at[pl.ds(off, C)], dst_v)
        pltpu.sync_copy(e_hbm.at[pl.ds(off, C)], e_v)
        pltpu.async_copy(z_hbm.at[src_v], zs_v, sem).wait()

        def group(g, _):
            rows = g * 16 + _iota16()
            d16 = plsc.load_gather(dst_v, [rows])
            e16 = plsc.load_gather(e_v, [rows])
            p16 = jnp.exp(e16 - gv)
            den16 = plsc.load_gather(den_v, [d16])
            plsc.store_scatter(a_v, [rows], p16 / den16)
            return 0
        lax.fori_loop(0, C // 16, group, 0)

        def edge(j, _):
            aj = plsc.load_gather(a_v, [jnp.full((16,), 1, jnp.int32) * j])
            for k in range(D // 16):
                sl = pl.ds(16 * k, 16)
                zs_v[j, sl] = zs_v[j, sl] * aj
            return 0
        lax.fori_loop(0, C, edge, 0)
        pltpu.sync_copy(zs_v, hout_sh.at[dst_v], add=True)
        return 0
    lax.fori_loop(0, NCH, chunk, 0)

    plsc.subcore_barrier()

    @pl.when(sid < NS - 1)
    def _():
        pltpu.sync_copy(hout_sh.at[pl.ds(s0, 624)],
                        out_hbm.at[cid, pl.ds(s0, 624)])

    @pl.when(sid == NS - 1)
    def _():
        pltpu.sync_copy(hout_sh.at[pl.ds(624 * (NS - 1), 640)],
                        out_hbm.at[cid, pl.ds(624 * (NS - 1), 640)])


def _aggregate(z, src, dst, e, mx, den, zeros):
    f = pl.kernel(
        _sc_aggregate_body,
        out_type=[jax.ShapeDtypeStruct((NC, N, D), jnp.float32)],
        mesh=plsc.VectorSubcoreMesh(core_axis_name="c", subcore_axis_name="s"),
        compiler_params=pltpu.CompilerParams(needs_layout_passes=False),
        scratch_types=[
            pltpu.VMEM((C,), jnp.int32),
            pltpu.VMEM((C,), jnp.int32),
            pltpu.VMEM((C,), jnp.float32),
            pltpu.VMEM((C,), jnp.float32),
            pltpu.VMEM((NW, 16), jnp.float32),
            pltpu.VMEM((N,), jnp.float32),
            pltpu.VMEM((C, D), jnp.float32),
            pltpu.VMEM_SHARED((N, D), jnp.float32),
            pltpu.SemaphoreType.DMA,
        ],
    )
    return f(z, src, dst, e, mx, den, zeros)[0]


@jax.jit
def _gat(h, edge_index, W):
    z = _project(h, W)
    src = edge_index[0].astype(jnp.int32)
    dst = edge_index[1].astype(jnp.int32)
    e, mx = _edge_logits(z, src, dst)
    den = _denom_total(_denom(dst, e, mx))
    zeros = jnp.zeros((N, D), jnp.float32)
    parts = _aggregate(z, src, dst, e, mx, den, zeros)
    return _finish(parts)


def kernel(h, edge_index, W):
    return _gat(h, edge_index, W)


# speedup vs baseline: 1.8498x; 1.8498x over previous
import jax
import jax.numpy as jnp
from jax import lax
from jax.experimental import pallas as pl
from jax.experimental.pallas import tpu as pltpu
from jax.experimental.pallas import tpu_sc as plsc

N = 10000
E = 320000
D = 128
NC = 2
NS = 16
NW = NC * NS
EW = E // NW
C = 80
NCH = EW // C
RPT = N // NS
BR = 400


def _mm_body(h_ref, w_ref, o_ref):
    o_ref[...] = jnp.dot(h_ref[...], w_ref[...], preferred_element_type=jnp.float32)


def _project(h, W):
    return pl.pallas_call(
        _mm_body,
        grid=(N // BR,),
        in_specs=[pl.BlockSpec((BR, D), lambda i: (i, 0)),
                  pl.BlockSpec((D, D), lambda i: (0, 0))],
        out_specs=pl.BlockSpec((BR, D), lambda i: (i, 0)),
        out_shape=jax.ShapeDtypeStruct((N, D), jnp.float32),
    )(h, W)


def _elu_body(p_ref, o_ref):
    s = p_ref[0] + p_ref[1]
    o_ref[...] = jnp.where(s > 0.0, s, jnp.exp(jnp.minimum(s, 0.0)) - 1.0)


def _sum_body(d_ref, o_ref):
    o_ref[...] = jnp.sum(d_ref[...], axis=0, keepdims=True)


def _denom_total(den):
    out = pl.pallas_call(
        _sum_body,
        out_shape=jax.ShapeDtypeStruct((1, N), jnp.float32),
    )(den)
    return out.reshape(N)


def _finish(parts):
    return pl.pallas_call(
        _elu_body,
        grid=(N // BR,),
        in_specs=[pl.BlockSpec((NC, BR, D), lambda i: (0, i, 0))],
        out_specs=pl.BlockSpec((BR, D), lambda i: (i, 0)),
        out_shape=jax.ShapeDtypeStruct((N, D), jnp.float32),
    )(parts)


def _iota16():
    return lax.iota(jnp.int32, 16)


def _take(x, idx):
    return x.at[idx].get(mode="promise_in_bounds")


def _worker_id():
    return lax.axis_index("s") * NC + lax.axis_index("c")


def _global_max_vec(mx_hbm, mx_v):
    pltpu.sync_copy(mx_hbm, mx_v)
    m = mx_v[0, pl.ds(0, 16)]
    for w in range(1, NW):
        m = jnp.maximum(m, mx_v[w, pl.ds(0, 16)])
    return jnp.full((16,), jnp.max(m), jnp.float32)


def _zero_1d(ref, n):
    zeros = jnp.zeros((16,), jnp.float32)

    def body(i, _):
        plsc.store_scatter(ref, [i * 16 + _iota16()], zeros)
        return 0
    lax.fori_loop(0, n // 16, body, 0)


def _sc_edge_logits_body(z_hbm, src_hbm, dst_hbm, e_hbm, mx_hbm,
                         src_v, dst_v, zs_v, zd_v, e_v, mxo_v, sem):
    base = _worker_id() * EW

    def chunk(i, rm):
        off = base + i * C
        pltpu.sync_copy(src_hbm.at[pl.ds(off, C)], src_v)
        pltpu.sync_copy(dst_hbm.at[pl.ds(off, C)], dst_v)
        d1 = pltpu.async_copy(z_hbm.at[src_v], zs_v, sem)
        d2 = pltpu.async_copy(z_hbm.at[dst_v], zd_v, sem)
        d1.wait()
        d2.wait()

        def group(g, rm):
            e16 = jnp.zeros((16,), jnp.float32)
            for jj in range(16):
                j = g * 16 + jj
                acc = zs_v[j, pl.ds(0, 16)] * zd_v[j, pl.ds(0, 16)]
                for k in range(1, D // 16):
                    acc = acc + (zs_v[j, pl.ds(16 * k, 16)] *
                                 zd_v[j, pl.ds(16 * k, 16)])
                dj = jnp.sum(acc)
                dj = jnp.where(dj >= 0.0, dj, 0.2 * dj)
                e16 = jnp.where(_iota16() == jj,
                                jnp.full((16,), dj, jnp.float32), e16)
            plsc.store_scatter(e_v, [g * 16 + _iota16()], e16)
            return jnp.maximum(rm, e16)

        rm = lax.fori_loop(0, C // 16, group, rm)
        pltpu.sync_copy(e_v, e_hbm.at[pl.ds(off, C)])
        return rm

    rm = lax.fori_loop(0, NCH, chunk, jnp.full((16,), -3.4e38, jnp.float32))
    mxo_v[...] = rm
    pltpu.sync_copy(mxo_v, mx_hbm.at[_worker_id()])


def _edge_logits(z, src, dst):
    f = pl.kernel(
        _sc_edge_logits_body,
        out_type=[jax.ShapeDtypeStruct((E,), jnp.float32),
                  jax.ShapeDtypeStruct((NW, 16), jnp.float32)],
        mesh=plsc.VectorSubcoreMesh(core_axis_name="c", subcore_axis_name="s"),
        compiler_params=pltpu.CompilerParams(needs_layout_passes=False),
        scratch_types=[
            pltpu.VMEM((C,), jnp.int32),
            pltpu.VMEM((C,), jnp.int32),
            pltpu.VMEM((C, D), jnp.float32),
            pltpu.VMEM((C, D), jnp.float32),
            pltpu.VMEM((C,), jnp.float32),
            pltpu.VMEM((16,), jnp.float32),
            pltpu.SemaphoreType.DMA,
        ],
    )
    return f(z, src, dst)


def _segsum_scatter_add(den_ref, keys, vals):
    sk, sv = plsc.sort_key_val(keys, vals)
    io = _iota16()
    for d in (1, 2, 4, 8):
        idx = jnp.maximum(io - d, 0)
        same = (io >= d) & (_take(sk, idx) == sk)
        sv = sv + jnp.where(same, _take(sv, idx), 0.0)
    nxt = _take(sk, jnp.minimum(io + 1, 15))
    last = (io == 15) | (nxt != sk)
    plsc.addupdate_scatter(den_ref, [sk], sv, mask=last)


def _sc_denom_body(dst_hbm, e_hbm, mx_hbm, den_hbm,
                   dst_v, e_v, mx_v, den_v):
    base = _worker_id() * EW
    gv = _global_max_vec(mx_hbm, mx_v)
    _zero_1d(den_v, N)

    def chunk(i, _):
        off = base + i * C
        pltpu.sync_copy(dst_hbm.at[pl.ds(off, C)], dst_v)
        pltpu.sync_copy(e_hbm.at[pl.ds(off, C)], e_v)

        def group(g, _):
            rows = g * 16 + _iota16()
            d16 = plsc.load_gather(dst_v, [rows])
            e16 = plsc.load_gather(e_v, [rows])
            p16 = jnp.exp(e16 - gv)
            _segsum_scatter_add(den_v, d16, p16)
            return 0
        lax.fori_loop(0, C // 16, group, 0)
        return 0
    lax.fori_loop(0, NCH, chunk, 0)

    pltpu.sync_copy(den_v, den_hbm.at[_worker_id()])


def _denom(dst, e, mx):
    f = pl.kernel(
        _sc_denom_body,
        out_type=[jax.ShapeDtypeStruct((NW, N), jnp.float32)],
        mesh=plsc.VectorSubcoreMesh(core_axis_name="c", subcore_axis_name="s"),
        compiler_params=pltpu.CompilerParams(needs_layout_passes=False),
        scratch_types=[
            pltpu.VMEM((C,), jnp.int32),
            pltpu.VMEM((C,), jnp.float32),
            pltpu.VMEM((NW, 16), jnp.float32),
            pltpu.VMEM((N,), jnp.float32),
        ],
    )
    return f(dst, e, mx)[0]


def _sc_aggregate_body(z_hbm, src_hbm, dst_hbm, e_hbm, mx_hbm, den_hbm, zero_hbm,
                       out_hbm,
                       src_v, dst_v, e_v, a_v, mx_v, den_v, zs_v,
                       hout_sh, sem):
    cid = lax.axis_index("c")
    sid = lax.axis_index("s")
    base = _worker_id() * EW
    gv = _global_max_vec(mx_hbm, mx_v)

    pltpu.sync_copy(den_hbm, den_v)

    s0 = pl.multiple_of(sid * 624, 8)

    @pl.when(sid < NS - 1)
    def _():
        pltpu.sync_copy(zero_hbm.at[pl.ds(s0, 624)],
                        hout_sh.at[pl.ds(s0, 624)])

    @pl.when(sid == NS - 1)
    def _():
        pltpu.sync_copy(zero_hbm.at[pl.ds(624 * (NS - 1), 640)],
                        hout_sh.at[pl.ds(624 * (NS - 1), 640)])
    plsc.subcore_barrier()

    def chunk(i, _):
        off = base + i * C
        pltpu.sync_copy(src_hbm.at[pl.ds(off, C)], src_v)
        pltpu.sync_copy(dst_hbm.at[pl.ds(off, C)], dst_v)
        pltpu.sync_copy(e_hbm.at[pl.ds(off, C)], e_v)
        pltpu.async_copy(z_hbm.at[src_v], zs_v, sem).wait()

        def group(g, _):
            rows = g * 16 + _iota16()
            d16 = plsc.load_gather(dst_v, [rows])
            e16 = plsc.load_gather(e_v, [rows])
            p16 = jnp.exp(e16 - gv)
            den16 = plsc.load_gather(den_v, [d16])
            plsc.store_scatter(a_v, [rows], p16 / den16)
            return 0
        lax.fori_loop(0, C // 16, group, 0)

        def edge(j, _):
            aj = plsc.load_gather(a_v, [jnp.full((16,), 1, jnp.int32) * j])
            for k in range(D // 16):
                sl = pl.ds(16 * k, 16)
                zs_v[j, sl] = zs_v[j, sl] * aj
            return 0
        lax.fori_loop(0, C, edge, 0)
        pltpu.sync_copy(zs_v, hout_sh.at[dst_v], add=True)
        return 0
    lax.fori_loop(0, NCH, chunk, 0)

    plsc.subcore_barrier()

    @pl.when(sid < NS - 1)
    def _():
        pltpu.sync_copy(hout_sh.at[pl.ds(s0, 624)],
                        out_hbm.at[cid, pl.ds(s0, 624)])

    @pl.when(sid == NS - 1)
    def _():
        pltpu.sync_copy(hout_sh.at[pl.ds(624 * (NS - 1), 640)],
                        out_hbm.at[cid, pl.ds(624 * (NS - 1), 640)])


def _aggregate(z, src, dst, e, mx, den, zeros):
    f = pl.kernel(
        _sc_aggregate_body,
        out_type=[jax.ShapeDtypeStruct((NC, N, D), jnp.float32)],
        mesh=plsc.VectorSubcoreMesh(core_axis_name="c", subcore_axis_name="s"),
        compiler_params=pltpu.CompilerParams(needs_layout_passes=False),
        scratch_types=[
            pltpu.VMEM((C,), jnp.int32),
            pltpu.VMEM((C,), jnp.int32),
            pltpu.VMEM((C,), jnp.float32),
            pltpu.VMEM((C,), jnp.float32),
            pltpu.VMEM((NW, 16), jnp.float32),
            pltpu.VMEM((N,), jnp.float32),
            pltpu.VMEM((C, D), jnp.float32),
            pltpu.VMEM_SHARED((N, D), jnp.float32),
            pltpu.SemaphoreType.DMA,
        ],
    )
    return f(z, src, dst, e, mx, den, zeros)[0]


@jax.jit
def _gat(h, edge_index, W):
    z = _project(h, W)
    src = edge_index[0].astype(jnp.int32)
    dst = edge_index[1].astype(jnp.int32)
    e, mx = _edge_logits(z, src, dst)
    den = _denom_total(_denom(dst, e, mx))
    zeros = jnp.zeros((N, D), jnp.float32)
    parts = _aggregate(z, src, dst, e, mx, den, zeros)
    return _finish(parts)


def kernel(h, edge_index, W):
    return _gat(h, edge_index, W)


# speedup vs baseline: 2.3819x; 1.2877x over previous
import jax
import jax.numpy as jnp
from jax import lax
from jax.experimental import pallas as pl
from jax.experimental.pallas import tpu as pltpu
from jax.experimental.pallas import tpu_sc as plsc

N = 10000
E = 320000
D = 128
NC = 2
NS = 16
NW = NC * NS
EW = E // NW
C = 80
NCH = EW // C
RPT = N // NS
BR = 400


def _mm_body(h_ref, w_ref, o_ref):
    o_ref[...] = jnp.dot(h_ref[...], w_ref[...], preferred_element_type=jnp.float32)


def _project(h, W):
    return pl.pallas_call(
        _mm_body,
        grid=(N // BR,),
        in_specs=[pl.BlockSpec((BR, D), lambda i: (i, 0)),
                  pl.BlockSpec((D, D), lambda i: (0, 0))],
        out_specs=pl.BlockSpec((BR, D), lambda i: (i, 0)),
        out_shape=jax.ShapeDtypeStruct((N, D), jnp.float32),
    )(h, W)


def _elu_body(p_ref, o_ref):
    s = p_ref[0] + p_ref[1]
    o_ref[...] = jnp.where(s > 0.0, s, jnp.exp(jnp.minimum(s, 0.0)) - 1.0)


def _sum_body(d_ref, o_ref):
    o_ref[...] = jnp.sum(d_ref[...], axis=0, keepdims=True)


def _denom_total(den):
    out = pl.pallas_call(
        _sum_body,
        out_shape=jax.ShapeDtypeStruct((1, N), jnp.float32),
    )(den)
    return out.reshape(N)


def _finish(parts):
    return pl.pallas_call(
        _elu_body,
        grid=(N // BR,),
        in_specs=[pl.BlockSpec((NC, BR, D), lambda i: (0, i, 0))],
        out_specs=pl.BlockSpec((BR, D), lambda i: (i, 0)),
        out_shape=jax.ShapeDtypeStruct((N, D), jnp.float32),
    )(parts)


def _iota16():
    return lax.iota(jnp.int32, 16)


def _take(x, idx):
    return x.at[idx].get(mode="promise_in_bounds")


def _worker_id():
    return lax.axis_index("s") * NC + lax.axis_index("c")


def _global_max_vec(mx_hbm, mx_v):
    pltpu.sync_copy(mx_hbm, mx_v)
    m = mx_v[0, pl.ds(0, 16)]
    for w in range(1, NW):
        m = jnp.maximum(m, mx_v[w, pl.ds(0, 16)])
    return jnp.full((16,), jnp.max(m), jnp.float32)


def _zero_1d(ref, n):
    zeros = jnp.zeros((16,), jnp.float32)

    def body(i, _):
        plsc.store_scatter(ref, [i * 16 + _iota16()], zeros)
        return 0
    lax.fori_loop(0, n // 16, body, 0)


def _sc_edge_logits_body(z_hbm, src_hbm, dst_hbm, e_hbm, mx_hbm,
                         src_v, dst_v, zs_v, zd_v, e_v, mxo_v, sem):
    base = _worker_id() * EW

    def fetch(ch, b):
        off = base + ch * C
        pltpu.sync_copy(src_hbm.at[pl.ds(off, C)], src_v.at[b])
        pltpu.sync_copy(dst_hbm.at[pl.ds(off, C)], dst_v.at[b])
        pltpu.async_copy(z_hbm.at[src_v.at[b]], zs_v.at[b], sem)
        pltpu.async_copy(z_hbm.at[dst_v.at[b]], zd_v.at[b], sem)

    def wait(b):
        pltpu.make_async_copy(z_hbm.at[src_v.at[b]], zs_v.at[b], sem).wait()
        pltpu.make_async_copy(z_hbm.at[dst_v.at[b]], zd_v.at[b], sem).wait()

    def compute(ch, b, rm):
        zsb = zs_v.at[b]
        zdb = zd_v.at[b]

        def group(g, rm):
            e16 = jnp.zeros((16,), jnp.float32)
            for jj in range(16):
                j = g * 16 + jj
                acc = zsb[j, pl.ds(0, 16)] * zdb[j, pl.ds(0, 16)]
                for k in range(1, D // 16):
                    acc = acc + (zsb[j, pl.ds(16 * k, 16)] *
                                 zdb[j, pl.ds(16 * k, 16)])
                dj = jnp.sum(acc)
                dj = jnp.where(dj >= 0.0, dj, 0.2 * dj)
                e16 = jnp.where(_iota16() == jj,
                                jnp.full((16,), dj, jnp.float32), e16)
            plsc.store_scatter(e_v, [g * 16 + _iota16()], e16)
            return jnp.maximum(rm, e16)

        rm = lax.fori_loop(0, C // 16, group, rm)
        pltpu.sync_copy(e_v, e_hbm.at[pl.ds(base + ch * C, C)])
        return rm

    fetch(0, 0)

    def pair(i2, rm):
        ch0 = i2 * 2
        wait(0)
        fetch(ch0 + 1, 1)
        rm = compute(ch0, 0, rm)
        wait(1)
        fetch(ch0 + 2, 0)
        rm = compute(ch0 + 1, 1, rm)
        return rm

    rm = lax.fori_loop(0, (NCH - 1) // 2, pair,
                       jnp.full((16,), -3.4e38, jnp.float32))
    wait(0)
    rm = compute(NCH - 1, 0, rm)
    mxo_v[...] = rm
    pltpu.sync_copy(mxo_v, mx_hbm.at[_worker_id()])


def _edge_logits(z, src, dst):
    f = pl.kernel(
        _sc_edge_logits_body,
        out_type=[jax.ShapeDtypeStruct((E,), jnp.float32),
                  jax.ShapeDtypeStruct((NW, 16), jnp.float32)],
        mesh=plsc.VectorSubcoreMesh(core_axis_name="c", subcore_axis_name="s"),
        compiler_params=pltpu.CompilerParams(needs_layout_passes=False),
        scratch_types=[
            pltpu.VMEM((2, C), jnp.int32),
            pltpu.VMEM((2, C), jnp.int32),
            pltpu.VMEM((2, C, D), jnp.float32),
            pltpu.VMEM((2, C, D), jnp.float32),
            pltpu.VMEM((C,), jnp.float32),
            pltpu.VMEM((16,), jnp.float32),
            pltpu.SemaphoreType.DMA,
        ],
    )
    return f(z, src, dst)


def _segsum_scatter_add(den_ref, keys, vals):
    sk, sv = plsc.sort_key_val(keys, vals)
    io = _iota16()
    for d in (1, 2, 4, 8):
        idx = jnp.maximum(io - d, 0)
        same = (io >= d) & (_take(sk, idx) == sk)
        sv = sv + jnp.where(same, _take(sv, idx), 0.0)
    nxt = _take(sk, jnp.minimum(io + 1, 15))
    last = (io == 15) | (nxt != sk)
    plsc.addupdate_scatter(den_ref, [sk], sv, mask=last)


def _sc_denom_body(dst_hbm, e_hbm, mx_hbm, den_hbm,
                   dst_v, e_v, mx_v, den_v):
    base = _worker_id() * EW
    gv = _global_max_vec(mx_hbm, mx_v)
    _zero_1d(den_v, N)

    def chunk(i, _):
        off = base + i * C
        pltpu.sync_copy(dst_hbm.at[pl.ds(off, C)], dst_v)
        pltpu.sync_copy(e_hbm.at[pl.ds(off, C)], e_v)

        def group(g, _):
            rows = g * 16 + _iota16()
            d16 = plsc.load_gather(dst_v, [rows])
            e16 = plsc.load_gather(e_v, [rows])
            p16 = jnp.exp(e16 - gv)
            _segsum_scatter_add(den_v, d16, p16)
            return 0
        lax.fori_loop(0, C // 16, group, 0)
        return 0
    lax.fori_loop(0, NCH, chunk, 0)

    pltpu.sync_copy(den_v, den_hbm.at[_worker_id()])


def _denom(dst, e, mx):
    f = pl.kernel(
        _sc_denom_body,
        out_type=[jax.ShapeDtypeStruct((NW, N), jnp.float32)],
        mesh=plsc.VectorSubcoreMesh(core_axis_name="c", subcore_axis_name="s"),
        compiler_params=pltpu.CompilerParams(needs_layout_passes=False),
        scratch_types=[
            pltpu.VMEM((C,), jnp.int32),
            pltpu.VMEM((C,), jnp.float32),
            pltpu.VMEM((NW, 16), jnp.float32),
            pltpu.VMEM((N,), jnp.float32),
        ],
    )
    return f(dst, e, mx)[0]


def _sc_aggregate_body(z_hbm, src_hbm, dst_hbm, e_hbm, mx_hbm, den_hbm, zero_hbm,
                       out_hbm,
                       src_v, dst_v, e_v, a_v, mx_v, den_v, zs_v,
                       hout_sh, sem):
    cid = lax.axis_index("c")
    sid = lax.axis_index("s")
    base = _worker_id() * EW
    gv = _global_max_vec(mx_hbm, mx_v)

    pltpu.sync_copy(den_hbm, den_v)

    s0 = pl.multiple_of(sid * 624, 8)

    @pl.when(sid < NS - 1)
    def _():
        pltpu.sync_copy(zero_hbm.at[pl.ds(s0, 624)],
                        hout_sh.at[pl.ds(s0, 624)])

    @pl.when(sid == NS - 1)
    def _():
        pltpu.sync_copy(zero_hbm.at[pl.ds(624 * (NS - 1), 640)],
                        hout_sh.at[pl.ds(624 * (NS - 1), 640)])
    plsc.subcore_barrier()

    def fetch(ch, b):
        off = base + ch * C
        pltpu.sync_copy(src_hbm.at[pl.ds(off, C)], src_v.at[b])
        pltpu.sync_copy(dst_hbm.at[pl.ds(off, C)], dst_v.at[b])
        pltpu.sync_copy(e_hbm.at[pl.ds(off, C)], e_v.at[b])
        pltpu.async_copy(z_hbm.at[src_v.at[b]], zs_v.at[b], sem)

    def wait(b):
        pltpu.make_async_copy(z_hbm.at[src_v.at[b]], zs_v.at[b], sem).wait()

    def compute(b):
        zsb = zs_v.at[b]

        def group(g, _):
            rows = g * 16 + _iota16()
            d16 = plsc.load_gather(dst_v.at[b], [rows])
            e16 = plsc.load_gather(e_v.at[b], [rows])
            p16 = jnp.exp(e16 - gv)
            den16 = plsc.load_gather(den_v, [d16])
            plsc.store_scatter(a_v, [rows], p16 / den16)
            return 0
        lax.fori_loop(0, C // 16, group, 0)

        def edge(j, _):
            aj = plsc.load_gather(a_v, [jnp.full((16,), 1, jnp.int32) * j])
            for k in range(D // 16):
                sl = pl.ds(16 * k, 16)
                zsb[j, sl] = zsb[j, sl] * aj
            return 0
        lax.fori_loop(0, C, edge, 0)
        pltpu.sync_copy(zsb, hout_sh.at[dst_v.at[b]], add=True)

    fetch(0, 0)

    def pair(i2, _):
        ch0 = i2 * 2
        wait(0)
        fetch(ch0 + 1, 1)
        compute(0)
        wait(1)
        fetch(ch0 + 2, 0)
        compute(1)
        return 0
    lax.fori_loop(0, (NCH - 1) // 2, pair, 0)
    wait(0)
    compute(0)

    plsc.subcore_barrier()

    @pl.when(sid < NS - 1)
    def _():
        pltpu.sync_copy(hout_sh.at[pl.ds(s0, 624)],
                        out_hbm.at[cid, pl.ds(s0, 624)])

    @pl.when(sid == NS - 1)
    def _():
        pltpu.sync_copy(hout_sh.at[pl.ds(624 * (NS - 1), 640)],
                        out_hbm.at[cid, pl.ds(624 * (NS - 1), 640)])


def _aggregate(z, src, dst, e, mx, den, zeros):
    f = pl.kernel(
        _sc_aggregate_body,
        out_type=[jax.ShapeDtypeStruct((NC, N, D), jnp.float32)],
        mesh=plsc.VectorSubcoreMesh(core_axis_name="c", subcore_axis_name="s"),
        compiler_params=pltpu.CompilerParams(needs_layout_passes=False),
        scratch_types=[
            pltpu.VMEM((2, C), jnp.int32),
            pltpu.VMEM((2, C), jnp.int32),
            pltpu.VMEM((2, C), jnp.float32),
            pltpu.VMEM((C,), jnp.float32),
            pltpu.VMEM((NW, 16), jnp.float32),
            pltpu.VMEM((N,), jnp.float32),
            pltpu.VMEM((2, C, D), jnp.float32),
            pltpu.VMEM_SHARED((N, D), jnp.float32),
            pltpu.SemaphoreType.DMA,
        ],
    )
    return f(z, src, dst, e, mx, den, zeros)[0]


@jax.jit
def _gat(h, edge_index, W):
    z = _project(h, W)
    src = edge_index[0].astype(jnp.int32)
    dst = edge_index[1].astype(jnp.int32)
    e, mx = _edge_logits(z, src, dst)
    den = _denom_total(_denom(dst, e, mx))
    zeros = jnp.zeros((N, D), jnp.float32)
    parts = _aggregate(z, src, dst, e, mx, den, zeros)
    return _finish(parts)


def kernel(h, edge_index, W):
    return _gat(h, edge_index, W)


# speedup vs baseline: 2.7346x; 1.1481x over previous
import jax
import jax.numpy as jnp
from jax import lax
from jax.experimental import pallas as pl
from jax.experimental.pallas import tpu as pltpu
from jax.experimental.pallas import tpu_sc as plsc

N = 10000
E = 320000
D = 128
NC = 2
NS = 16
NW = NC * NS
EW = E // NW
C = 80
NCH = EW // C
RPT = N // NS
BR = 400


def _mm_body(h_ref, w_ref, o_ref):
    o_ref[...] = jnp.dot(h_ref[...], w_ref[...], preferred_element_type=jnp.float32)


def _project(h, W):
    return pl.pallas_call(
        _mm_body,
        grid=(N // BR,),
        in_specs=[pl.BlockSpec((BR, D), lambda i: (i, 0)),
                  pl.BlockSpec((D, D), lambda i: (0, 0))],
        out_specs=pl.BlockSpec((BR, D), lambda i: (i, 0)),
        out_shape=jax.ShapeDtypeStruct((N, D), jnp.float32),
    )(h, W)


def _elu_body(p_ref, o_ref):
    s = p_ref[0] + p_ref[1]
    o_ref[...] = jnp.where(s > 0.0, s, jnp.exp(jnp.minimum(s, 0.0)) - 1.0)


def _sum_body(d_ref, m_ref, o_ref):
    mw = jnp.max(m_ref[...], axis=1, keepdims=True)
    scale = jnp.exp(mw - jnp.max(mw))
    o_ref[...] = jnp.sum(d_ref[...] * scale, axis=0, keepdims=True)


def _denom_total(den, mx):
    out = pl.pallas_call(
        _sum_body,
        out_shape=jax.ShapeDtypeStruct((1, N), jnp.float32),
    )(den, mx)
    return out.reshape(N)


def _finish(parts):
    return pl.pallas_call(
        _elu_body,
        grid=(N // BR,),
        in_specs=[pl.BlockSpec((NC, BR, D), lambda i: (0, i, 0))],
        out_specs=pl.BlockSpec((BR, D), lambda i: (i, 0)),
        out_shape=jax.ShapeDtypeStruct((N, D), jnp.float32),
    )(parts)


def _iota16():
    return lax.iota(jnp.int32, 16)


def _take(x, idx):
    return x.at[idx].get(mode="promise_in_bounds")


def _worker_id():
    return lax.axis_index("s") * NC + lax.axis_index("c")


def _global_max_vec(mx_hbm, mx_v):
    pltpu.sync_copy(mx_hbm, mx_v)
    m = mx_v[0, pl.ds(0, 16)]
    for w in range(1, NW):
        m = jnp.maximum(m, mx_v[w, pl.ds(0, 16)])
    return jnp.full((16,), jnp.max(m), jnp.float32)


def _zero_1d(ref, n):
    zeros = jnp.zeros((16,), jnp.float32)

    def body(i, _):
        plsc.store_scatter(ref, [i * 16 + _iota16()], zeros)
        return 0
    lax.fori_loop(0, n // 16, body, 0)


def _sc_edge_logits_body(z_hbm, src_hbm, dst_hbm, e_hbm, mx_hbm, den_hbm,
                         src_v, dst_all, zs_v, zd_v, e_all, mxo_v, den_v, sem):
    base = _worker_id() * EW

    def fetch(ch, b):
        off = base + ch * C
        pltpu.sync_copy(src_hbm.at[pl.ds(off, C)], src_v.at[b])
        pltpu.sync_copy(dst_hbm.at[pl.ds(off, C)], dst_all.at[pl.ds(ch * C, C)])
        pltpu.async_copy(z_hbm.at[src_v.at[b]], zs_v.at[b], sem)
        pltpu.async_copy(z_hbm.at[dst_all.at[pl.ds(ch * C, C)]], zd_v.at[b], sem)

    def wait(ch, b):
        pltpu.make_async_copy(z_hbm.at[src_v.at[b]], zs_v.at[b], sem).wait()
        pltpu.make_async_copy(z_hbm.at[dst_all.at[pl.ds(ch * C, C)]],
                              zd_v.at[b], sem).wait()

    def compute(ch, b, rm):
        zsb = zs_v.at[b]
        zdb = zd_v.at[b]

        def group(g, rm):
            e16 = jnp.zeros((16,), jnp.float32)
            for jj in range(16):
                j = g * 16 + jj
                acc = zsb[j, pl.ds(0, 16)] * zdb[j, pl.ds(0, 16)]
                for k in range(1, D // 16):
                    acc = acc + (zsb[j, pl.ds(16 * k, 16)] *
                                 zdb[j, pl.ds(16 * k, 16)])
                dj = jnp.sum(acc)
                dj = jnp.where(dj >= 0.0, dj, 0.2 * dj)
                e16 = jnp.where(_iota16() == jj,
                                jnp.full((16,), dj, jnp.float32), e16)
            plsc.store_scatter(e_all, [ch * C + g * 16 + _iota16()], e16)
            return jnp.maximum(rm, e16)

        rm = lax.fori_loop(0, C // 16, group, rm)
        pltpu.sync_copy(e_all.at[pl.ds(ch * C, C)],
                        e_hbm.at[pl.ds(base + ch * C, C)])
        return rm

    fetch(0, 0)

    def pair(i2, rm):
        ch0 = i2 * 2
        wait(ch0, 0)
        fetch(ch0 + 1, 1)
        rm = compute(ch0, 0, rm)
        wait(ch0 + 1, 1)
        fetch(ch0 + 2, 0)
        rm = compute(ch0 + 1, 1, rm)
        return rm

    rm = lax.fori_loop(0, (NCH - 1) // 2, pair,
                       jnp.full((16,), -3.4e38, jnp.float32))
    wait(NCH - 1, 0)
    rm = compute(NCH - 1, 0, rm)
    mxo_v[...] = rm
    pltpu.sync_copy(mxo_v, mx_hbm.at[_worker_id()])

    m_loc = jnp.full((16,), jnp.max(rm), jnp.float32)
    _zero_1d(den_v, N)

    def dgroup(g, _):
        rows = g * 16 + _iota16()
        d16 = plsc.load_gather(dst_all, [rows])
        e16 = plsc.load_gather(e_all, [rows])
        _segsum_scatter_add(den_v, d16, jnp.exp(e16 - m_loc))
        return 0
    lax.fori_loop(0, EW // 16, dgroup, 0)
    pltpu.sync_copy(den_v, den_hbm.at[_worker_id()])


def _edge_logits(z, src, dst):
    f = pl.kernel(
        _sc_edge_logits_body,
        out_type=[jax.ShapeDtypeStruct((E,), jnp.float32),
                  jax.ShapeDtypeStruct((NW, 16), jnp.float32),
                  jax.ShapeDtypeStruct((NW, N), jnp.float32)],
        mesh=plsc.VectorSubcoreMesh(core_axis_name="c", subcore_axis_name="s"),
        compiler_params=pltpu.CompilerParams(needs_layout_passes=False),
        scratch_types=[
            pltpu.VMEM((2, C), jnp.int32),
            pltpu.VMEM((EW,), jnp.int32),
            pltpu.VMEM((2, C, D), jnp.float32),
            pltpu.VMEM((2, C, D), jnp.float32),
            pltpu.VMEM((EW,), jnp.float32),
            pltpu.VMEM((16,), jnp.float32),
            pltpu.VMEM((N,), jnp.float32),
            pltpu.SemaphoreType.DMA,
        ],
    )
    return f(z, src, dst)


def _segsum_scatter_add(den_ref, keys, vals):
    sk, sv = plsc.sort_key_val(keys, vals)
    io = _iota16()
    for d in (1, 2, 4, 8):
        idx = jnp.maximum(io - d, 0)
        same = (io >= d) & (_take(sk, idx) == sk)
        sv = sv + jnp.where(same, _take(sv, idx), 0.0)
    nxt = _take(sk, jnp.minimum(io + 1, 15))
    last = (io == 15) | (nxt != sk)
    plsc.addupdate_scatter(den_ref, [sk], sv, mask=last)


def _sc_aggregate_body(z_hbm, src_hbm, dst_hbm, e_hbm, mx_hbm, den_hbm, zero_hbm,
                       out_hbm,
                       src_v, dst_v, e_v, a_v, mx_v, den_v, zs_v,
                       hout_sh, sem):
    cid = lax.axis_index("c")
    sid = lax.axis_index("s")
    base = _worker_id() * EW
    gv = _global_max_vec(mx_hbm, mx_v)

    pltpu.sync_copy(den_hbm, den_v)

    s0 = pl.multiple_of(sid * 624, 8)

    @pl.when(sid < NS - 1)
    def _():
        pltpu.sync_copy(zero_hbm.at[pl.ds(s0, 624)],
                        hout_sh.at[pl.ds(s0, 624)])

    @pl.when(sid == NS - 1)
    def _():
        pltpu.sync_copy(zero_hbm.at[pl.ds(624 * (NS - 1), 640)],
                        hout_sh.at[pl.ds(624 * (NS - 1), 640)])
    plsc.subcore_barrier()

    def fetch(ch, b):
        off = base + ch * C
        pltpu.sync_copy(src_hbm.at[pl.ds(off, C)], src_v.at[b])
        pltpu.sync_copy(dst_hbm.at[pl.ds(off, C)], dst_v.at[b])
        pltpu.sync_copy(e_hbm.at[pl.ds(off, C)], e_v.at[b])
        pltpu.async_copy(z_hbm.at[src_v.at[b]], zs_v.at[b], sem)

    def wait(b):
        pltpu.make_async_copy(z_hbm.at[src_v.at[b]], zs_v.at[b], sem).wait()

    def compute(b):
        zsb = zs_v.at[b]

        def group(g, _):
            rows = g * 16 + _iota16()
            d16 = plsc.load_gather(dst_v.at[b], [rows])
            e16 = plsc.load_gather(e_v.at[b], [rows])
            p16 = jnp.exp(e16 - gv)
            den16 = plsc.load_gather(den_v, [d16])
            plsc.store_scatter(a_v, [rows], p16 / den16)
            return 0
        lax.fori_loop(0, C // 16, group, 0)

        def edge(j, _):
            aj = plsc.load_gather(a_v, [jnp.full((16,), 1, jnp.int32) * j])
            for k in range(D // 16):
                sl = pl.ds(16 * k, 16)
                zsb[j, sl] = zsb[j, sl] * aj
            return 0
        lax.fori_loop(0, C, edge, 0)
        pltpu.sync_copy(zsb, hout_sh.at[dst_v.at[b]], add=True)

    fetch(0, 0)

    def pair(i2, _):
        ch0 = i2 * 2
        wait(0)
        fetch(ch0 + 1, 1)
        compute(0)
        wait(1)
        fetch(ch0 + 2, 0)
        compute(1)
        return 0
    lax.fori_loop(0, (NCH - 1) // 2, pair, 0)
    wait(0)
    compute(0)

    plsc.subcore_barrier()

    @pl.when(sid < NS - 1)
    def _():
        pltpu.sync_copy(hout_sh.at[pl.ds(s0, 624)],
                        out_hbm.at[cid, pl.ds(s0, 624)])

    @pl.when(sid == NS - 1)
    def _():
        pltpu.sync_copy(hout_sh.at[pl.ds(624 * (NS - 1), 640)],
                        out_hbm.at[cid, pl.ds(624 * (NS - 1), 640)])


def _aggregate(z, src, dst, e, mx, den, zeros):
    f = pl.kernel(
        _sc_aggregate_body,
        out_type=[jax.ShapeDtypeStruct((NC, N, D), jnp.float32)],
        mesh=plsc.VectorSubcoreMesh(core_axis_name="c", subcore_axis_name="s"),
        compiler_params=pltpu.CompilerParams(needs_layout_passes=False),
        scratch_types=[
            pltpu.VMEM((2, C), jnp.int32),
            pltpu.VMEM((2, C), jnp.int32),
            pltpu.VMEM((2, C), jnp.float32),
            pltpu.VMEM((C,), jnp.float32),
            pltpu.VMEM((NW, 16), jnp.float32),
            pltpu.VMEM((N,), jnp.float32),
            pltpu.VMEM((2, C, D), jnp.float32),
            pltpu.VMEM_SHARED((N, D), jnp.float32),
            pltpu.SemaphoreType.DMA,
        ],
    )
    return f(z, src, dst, e, mx, den, zeros)[0]


@jax.jit
def _gat(h, edge_index, W):
    z = _project(h, W)
    src = edge_index[0].astype(jnp.int32)
    dst = edge_index[1].astype(jnp.int32)
    e, mx, den = _edge_logits(z, src, dst)
    den = _denom_total(den, mx)
    zeros = jnp.zeros((N, D), jnp.float32)
    parts = _aggregate(z, src, dst, e, mx, den, zeros)
    return _finish(parts)


def kernel(h, edge_index, W):
    return _gat(h, edge_index, W)
